# baseline (speedup 1.0000x reference)
import functools
import math

import jax
import jax.numpy as jnp
from jax import lax
from jax.experimental import pallas as pl
from jax.experimental.pallas import tpu as pltpu
from jax.experimental.pallas import tpu_sc as plsc

S = 64.0
MARGIN = 0.5
SCOS = S * math.cos(MARGIN)
SSIN = S * math.sin(MARGIN)

NC = 2
NS = 16
NW = NC * NS
L = 16

CH = 20000
NBUF = 2


def _sc_body(B, V, logits_hbm, labels_hbm, out_hbm,
             in0, in1, o0, o1, lab_v, idx_v, x_v, v_v,
             si0, si1, so0, so1, gsem):
    pw = (B * V) // NW
    bpw = B // NW
    nch = pw // CH
    wid = lax.axis_index("s") * NC + lax.axis_index("c")
    base = wid * pw

    ins = (in0, in1)
    outs = (o0, o1)
    sis = (si0, si1)
    sos = (so0, so1)

    def load(c, b):
        pltpu.async_copy(logits_hbm.at[pl.ds(base + c * CH, CH)], ins[b], sis[b])

    def store(c, b):
        pltpu.async_copy(outs[b], out_hbm.at[pl.ds(base + c * CH, CH)], sos[b])

    load(0, 0)
    load(1, 1)

    def ring(gp, _):
        for b in range(NBUF):
            c = gp * NBUF + b

            @pl.when(gp >= 1)
            def _():
                pltpu.make_async_copy(
                    outs[b], out_hbm.at[pl.ds(base, CH)], sos[b]
                ).wait()

            pltpu.make_async_copy(
                logits_hbm.at[pl.ds(base, CH)], ins[b], sis[b]
            ).wait()

            @plsc.parallel_loop(0, CH, step=L, unroll=8)
            def _(i):
                outs[b][pl.ds(i, L)] = ins[b][pl.ds(i, L)] * S

            store(c, b)

            @pl.when(gp < nch // NBUF - 1)
            def _():
                load(c + NBUF, b)
        return 0

    lax.fori_loop(0, nch // NBUF, ring, 0)

    for b in range(NBUF):
        pltpu.make_async_copy(outs[b], out_hbm.at[pl.ds(base, CH)], sos[b]).wait()

    rbase = wid * bpw
    pltpu.sync_copy(labels_hbm.at[pl.ds(rbase, bpw)], lab_v)
    for c in range(bpw // L):
        lab = lab_v[pl.ds(c * L, L)]
        rows = rbase + c * L + lax.iota(jnp.int32, L)
        idx_v[pl.ds(c * L, L)] = rows * V + jnp.maximum(lab, 0)
    pltpu.async_copy(logits_hbm.at[idx_v], x_v, gsem).wait()
    for c in range(bpw // L):
        x = x_v[pl.ds(c * L, L)]
        lab = lab_v[pl.ds(c * L, L)]
        t = jnp.maximum(1.0 - x * x, 0.0)
        g = 0.5 * (1.0 + t)
        for _ in range(18):
            g = 0.5 * (g + t / g)
        v_v[pl.ds(c * L, L)] = jnp.where(lab >= 0, SCOS * x - SSIN * g, S * x)
    pltpu.async_copy(v_v, out_hbm.at[idx_v], gsem).wait()


@functools.lru_cache(maxsize=None)
def _make_sc_kernel(B, V):
    bpw = B // NW
    return functools.partial(
        pl.kernel,
        out_type=jax.ShapeDtypeStruct((B * V,), jnp.float32),
        mesh=plsc.VectorSubcoreMesh(core_axis_name="c", subcore_axis_name="s"),
        scratch_types=[
            pltpu.VMEM((CH,), jnp.float32),
            pltpu.VMEM((CH,), jnp.float32),
            pltpu.VMEM((CH,), jnp.float32),
            pltpu.VMEM((CH,), jnp.float32),
            pltpu.VMEM((bpw,), jnp.int32),
            pltpu.VMEM((bpw,), jnp.int32),
            pltpu.VMEM((bpw,), jnp.float32),
            pltpu.VMEM((bpw,), jnp.float32),
            pltpu.SemaphoreType.DMA,
            pltpu.SemaphoreType.DMA,
            pltpu.SemaphoreType.DMA,
            pltpu.SemaphoreType.DMA,
            pltpu.SemaphoreType.DMA,
        ],
    )(functools.partial(_sc_body, B, V))


def kernel(logits, labels):
    B, V = logits.shape
    labels = labels.astype(jnp.int32)
    out = _make_sc_kernel(B, V)(logits.reshape(B * V), labels)
    return out.reshape(B, V)


# speedup vs baseline: 1.7664x; 1.7664x over previous
import functools
import math

import jax
import jax.numpy as jnp
from jax import lax
from jax.experimental import pallas as pl
from jax.experimental.pallas import tpu as pltpu
from jax.experimental.pallas import tpu_sc as plsc

S = 64.0
MARGIN = 0.5
SCOS = S * math.cos(MARGIN)
SSIN = S * math.sin(MARGIN)

NC = 2
NS = 16
NW = NC * NS
L = 16

CW = 1408
NBUF = 2


def _margin_fix(in_buf, out_buf, lab_ref, band_local, col0, width):
    lane16 = lax.iota(jnp.int32, L)
    for r in range(8):
        lab = lab_ref[band_local * 8 + r]
        local = lab - col0
        hit = (local >= 0) & (local < width)

        @pl.when(hit)
        def _():
            start = pl.multiple_of((local >> 4) << 4, L)
            lane = local & 15
            y = in_buf[r, pl.ds(start, L)]
            t = jnp.maximum(1.0 - y * y, 0.0)
            g = 0.5 * (1.0 + t)
            for _ in range(18):
                g = 0.5 * (g + t / g)
            fixed = SCOS * y - SSIN * g
            out_buf[r, pl.ds(start, L)] = jnp.where(lane16 == lane, fixed, y * S)


def _sc_body(B, V, logits_hbm, labels_hbm, out_hbm,
             in0, in1, o0, o1, t_in, t_out, lab_v, smem_lab,
             si0, si1, so0, so1):
    bpw = B // NW
    nband = bpw // 8
    vmain = V - V % 128
    cpb = vmain // CW
    nch = nband * cpb
    wid = lax.axis_index("s") * NC + lax.axis_index("c")
    rbase = wid * bpw

    ins = (in0, in1)
    outs = (o0, o1)
    sis = (si0, si1)
    sos = (so0, so1)

    pltpu.sync_copy(labels_hbm.at[pl.ds(rbase, bpw)], lab_v)
    lab_lo = lab_v[pl.ds(0, L)]
    lab_hi = lab_v[pl.ds(L, L)]
    for j in range(L):
        smem_lab[j] = lab_lo[j]
        smem_lab[j + L] = lab_hi[j]

    def load(c, b):
        r0 = rbase + (c // cpb) * 8
        col = (c % cpb) * CW
        pltpu.async_copy(
            logits_hbm.at[pl.ds(r0, 8), pl.ds(col, CW)], ins[b], sis[b])

    def store(c, b):
        r0 = rbase + (c // cpb) * 8
        col = (c % cpb) * CW
        pltpu.async_copy(
            outs[b], out_hbm.at[pl.ds(r0, 8), pl.ds(col, CW)], sos[b])

    load(0, 0)
    load(1, 1)

    def ring(gp, _):
        for b in range(NBUF):
            c = gp * NBUF + b
            band_local = c // cpb
            col0 = (c % cpb) * CW

            @pl.when(gp >= 1)
            def _():
                pltpu.make_async_copy(
                    outs[b], out_hbm.at[pl.ds(rbase, 8), pl.ds(0, CW)], sos[b]
                ).wait()

            pltpu.make_async_copy(
                logits_hbm.at[pl.ds(rbase, 8), pl.ds(0, CW)], ins[b], sis[b]
            ).wait()

            for r in range(8):
                @plsc.parallel_loop(0, CW, step=L, unroll=8)
                def _(i):
                    outs[b][r, pl.ds(i, L)] = ins[b][r, pl.ds(i, L)] * S

            _margin_fix(ins[b], outs[b], smem_lab, band_local, col0, CW)

            store(c, b)

            @pl.when(gp < nch // NBUF - 1)
            def _():
                load(c + NBUF, b)
        return 0

    lax.fori_loop(0, nch // NBUF, ring, 0)

    for b in range(NBUF):
        pltpu.make_async_copy(
            outs[b], out_hbm.at[pl.ds(rbase, 8), pl.ds(0, CW)], sos[b]).wait()

    tail = V - vmain
    for band in range(nband):
        r0 = rbase + band * 8
        pltpu.sync_copy(logits_hbm.at[pl.ds(r0, 8), pl.ds(vmain, tail)], t_in)
        for r in range(8):
            @plsc.parallel_loop(0, tail, step=L)
            def _(i):
                t_out[r, pl.ds(i, L)] = t_in[r, pl.ds(i, L)] * S
        _margin_fix(t_in, t_out, smem_lab, band, vmain, tail)
        pltpu.sync_copy(t_out, out_hbm.at[pl.ds(r0, 8), pl.ds(vmain, tail)])


@functools.lru_cache(maxsize=None)
def _make_sc_kernel(B, V):
    bpw = B // NW
    tail = V % 128
    return functools.partial(
        pl.kernel,
        out_type=jax.ShapeDtypeStruct((B, V), jnp.float32),
        mesh=plsc.VectorSubcoreMesh(core_axis_name="c", subcore_axis_name="s"),
        scratch_types=[
            pltpu.VMEM((8, CW), jnp.float32),
            pltpu.VMEM((8, CW), jnp.float32),
            pltpu.VMEM((8, CW), jnp.float32),
            pltpu.VMEM((8, CW), jnp.float32),
            pltpu.VMEM((8, tail), jnp.float32),
            pltpu.VMEM((8, tail), jnp.float32),
            pltpu.VMEM((bpw,), jnp.int32),
            pltpu.SMEM((bpw,), jnp.int32),
            pltpu.SemaphoreType.DMA,
            pltpu.SemaphoreType.DMA,
            pltpu.SemaphoreType.DMA,
            pltpu.SemaphoreType.DMA,
        ],
    )(functools.partial(_sc_body, B, V))


def kernel(logits, labels):
    B, V = logits.shape
    labels = labels.astype(jnp.int32)
    return _make_sc_kernel(B, V)(logits, labels)


# speedup vs baseline: 6.2733x; 3.5514x over previous
import functools
import math

import jax
import jax.numpy as jnp
from jax import lax
from jax.experimental import pallas as pl
from jax.experimental.pallas import tpu as pltpu
from jax.experimental.pallas import tpu_sc as plsc

S = 64.0
MARGIN = 0.5
SCOS = S * math.cos(MARGIN)
SSIN = S * math.sin(MARGIN)

NC = 2
NS = 16
NW = NC * NS
L = 16

NCG = 8
NRG = 4
CH = 200
NBUF = 2


def _sc_body(B, V, xt_hbm, labels_hbm, out_hbm,
             in0, in1, o0, o1, lab_v, smem_lab,
             si0, si1, so0, so1):
    rpg = V // NRG
    nch = rpg // CH
    wid = lax.axis_index("s") * NC + lax.axis_index("c")
    cg = wid // NRG
    rg = wid % NRG
    col0 = cg * 128
    row_base = rg * rpg

    ins = (in0, in1)
    outs = (o0, o1)
    sis = (si0, si1)
    sos = (so0, so1)

    pltpu.sync_copy(labels_hbm.at[pl.ds(col0, 128)], lab_v)
    for g in range(8):
        vec = lab_v[pl.ds(g * L, L)]
        for j in range(L):
            smem_lab[g * L + j] = vec[j]

    lane16 = lax.iota(jnp.int32, L)

    def load(c, b):
        pltpu.async_copy(
            xt_hbm.at[pl.ds(row_base + c * CH, CH), pl.ds(col0, 128)],
            ins[b], sis[b])

    def store(c, b):
        pltpu.async_copy(
            outs[b], out_hbm.at[pl.ds(row_base + c * CH, CH), pl.ds(col0, 128)],
            sos[b])

    def wait_load(b):
        pltpu.make_async_copy(
            xt_hbm.at[pl.ds(row_base, CH), pl.ds(col0, 128)], ins[b], sis[b]
        ).wait()

    def wait_store(b):
        pltpu.make_async_copy(
            outs[b], out_hbm.at[pl.ds(row_base, CH), pl.ds(col0, 128)], sos[b]
        ).wait()

    def process(c, b, first):
        r0 = c * CH

        if not first:
            wait_store(b)
        wait_load(b)

        @plsc.parallel_loop(0, CH, unroll=2)
        def _(i):
            for k in range(8):
                outs[b][i, pl.ds(k * L, L)] = ins[b][i, pl.ds(k * L, L)] * S

        def fix(jj, carry):
            lab = smem_lab[jj]
            local = lab - (row_base + r0)
            hit = (local >= 0) & (local < CH)

            @pl.when(hit)
            def _():
                row = jnp.clip(local, 0, CH - 1)
                start = pl.multiple_of((jj >> 4) << 4, L)
                lane = jj & 15
                y = ins[b][row, pl.ds(start, L)]
                t = jnp.maximum(1.0 - y * y, 0.0)
                gg = 0.5 * (1.0 + t)
                for _ in range(18):
                    gg = 0.5 * (gg + t / gg)
                fixed = SCOS * y - SSIN * gg
                outs[b][row, pl.ds(start, L)] = jnp.where(
                    lane16 == lane, fixed, y * S)
            return carry

        lax.fori_loop(0, 128, fix, 0)

        store(c, b)

    load(0, 0)
    load(1, 1)
    process(0, 0, first=True)
    load(2, 0)
    process(1, 1, first=True)
    load(3, 1)

    def ring(gp, _):
        for b in range(NBUF):
            c = 2 + gp * NBUF + b
            process(c, b, first=False)

            @pl.when(c + NBUF <= nch - 1)
            def _():
                load(c + NBUF, b)
        return 0

    lax.fori_loop(0, (nch - 2) // NBUF, ring, 0)
    process(nch - 1, 0, first=False)

    wait_store(0)
    wait_store(1)


@functools.lru_cache(maxsize=None)
def _make_sc_kernel(B, V):
    return functools.partial(
        pl.kernel,
        out_type=jax.ShapeDtypeStruct((V, B), jnp.float32),
        mesh=plsc.VectorSubcoreMesh(core_axis_name="c", subcore_axis_name="s"),
        scratch_types=[
            pltpu.VMEM((CH, 128), jnp.float32),
            pltpu.VMEM((CH, 128), jnp.float32),
            pltpu.VMEM((CH, 128), jnp.float32),
            pltpu.VMEM((CH, 128), jnp.float32),
            pltpu.VMEM((128,), jnp.int32),
            pltpu.SMEM((128,), jnp.int32),
            pltpu.SemaphoreType.DMA,
            pltpu.SemaphoreType.DMA,
            pltpu.SemaphoreType.DMA,
            pltpu.SemaphoreType.DMA,
        ],
    )(functools.partial(_sc_body, B, V))


def kernel(logits, labels):
    B, V = logits.shape
    labels = labels.astype(jnp.int32)
    out_t = _make_sc_kernel(B, V)(logits.T, labels)
    return out_t.T


# speedup vs baseline: 6.2737x; 1.0001x over previous
import functools
import math

import jax
import jax.numpy as jnp
from jax import lax
from jax.experimental import pallas as pl
from jax.experimental.pallas import tpu as pltpu
from jax.experimental.pallas import tpu_sc as plsc

S = 64.0
MARGIN = 0.5
SCOS = S * math.cos(MARGIN)
SSIN = S * math.sin(MARGIN)

NC = 2
NS = 16
NW = NC * NS
L = 16

NCG = 8
NRG = 4
CH = 200
NBUF = 2


def _sc_body(B, V, xt_hbm, labels_hbm, out_hbm,
             in0, in1, o0, o1, lab_v, smem_lab,
             si0, si1, so0, so1):
    rpg = V // NRG
    nch = rpg // CH
    wid = lax.axis_index("s") * NC + lax.axis_index("c")
    cg = wid // NRG
    rg = wid % NRG
    col0 = cg * 128
    row_base = rg * rpg

    ins = (in0, in1)
    outs = (o0, o1)
    sis = (si0, si1)
    sos = (so0, so1)

    pltpu.sync_copy(labels_hbm.at[pl.ds(col0, 128)], lab_v)
    for g in range(8):
        vec = lab_v[pl.ds(g * L, L)]
        for j in range(L):
            smem_lab[g * L + j] = vec[j]

    lane16 = lax.iota(jnp.int32, L)

    def load(c, b):
        pltpu.async_copy(
            xt_hbm.at[pl.ds(row_base + c * CH, CH), pl.ds(col0, 128)],
            ins[b], sis[b])

    def store(c, b):
        pltpu.async_copy(
            outs[b], out_hbm.at[pl.ds(row_base + c * CH, CH), pl.ds(col0, 128)],
            sos[b])

    def wait_load(b):
        pltpu.make_async_copy(
            xt_hbm.at[pl.ds(row_base, CH), pl.ds(col0, 128)], ins[b], sis[b]
        ).wait()

    def wait_store(b):
        pltpu.make_async_copy(
            outs[b], out_hbm.at[pl.ds(row_base, CH), pl.ds(col0, 128)], sos[b]
        ).wait()

    def process(c, b, first):
        r0 = c * CH

        if not first:
            wait_store(b)
        wait_load(b)

        @plsc.parallel_loop(0, CH, unroll=2)
        def _(i):
            for k in range(8):
                outs[b][i, pl.ds(k * L, L)] = ins[b][i, pl.ds(k * L, L)] * S

        def fix(jj, carry):
            lab = smem_lab[jj]
            local = lab - (row_base + r0)
            hit = (local >= 0) & (local < CH)

            @pl.when(hit)
            def _():
                row = jnp.clip(local, 0, CH - 1)
                start = pl.multiple_of((jj >> 4) << 4, L)
                lane = jj & 15
                y = ins[b][row, pl.ds(start, L)]
                t = jnp.maximum(1.0 - y * y, 0.0)
                gg = 0.5 * (1.0 + t)
                for _ in range(18):
                    gg = 0.5 * (gg + t / gg)
                fixed = SCOS * y - SSIN * gg
                cur = outs[b][row, pl.ds(start, L)]
                outs[b][row, pl.ds(start, L)] = jnp.where(
                    lane16 == lane, fixed, cur)
            return carry

        lax.fori_loop(0, 128, fix, 0)

        store(c, b)

    load(0, 0)
    load(1, 1)
    process(0, 0, first=True)
    load(2, 0)
    process(1, 1, first=True)
    load(3, 1)

    def ring(gp, _):
        for b in range(NBUF):
            c = 2 + gp * NBUF + b
            process(c, b, first=False)

            @pl.when(c + NBUF <= nch - 1)
            def _():
                load(c + NBUF, b)
        return 0

    lax.fori_loop(0, (nch - 2) // NBUF, ring, 0)
    process(nch - 1, 0, first=False)

    wait_store(0)
    wait_store(1)


@functools.lru_cache(maxsize=None)
def _make_sc_kernel(B, V):
    return functools.partial(
        pl.kernel,
        out_type=jax.ShapeDtypeStruct((V, B), jnp.float32),
        mesh=plsc.VectorSubcoreMesh(core_axis_name="c", subcore_axis_name="s"),
        scratch_types=[
            pltpu.VMEM((CH, 128), jnp.float32),
            pltpu.VMEM((CH, 128), jnp.float32),
            pltpu.VMEM((CH, 128), jnp.float32),
            pltpu.VMEM((CH, 128), jnp.float32),
            pltpu.VMEM((128,), jnp.int32),
            pltpu.SMEM((128,), jnp.int32),
            pltpu.SemaphoreType.DMA,
            pltpu.SemaphoreType.DMA,
            pltpu.SemaphoreType.DMA,
            pltpu.SemaphoreType.DMA,
        ],
    )(functools.partial(_sc_body, B, V))


def kernel(logits, labels):
    B, V = logits.shape
    labels = labels.astype(jnp.int32)
    out_t = _make_sc_kernel(B, V)(logits.T, labels)
    return out_t.T
